# Initial kernel scaffold; baseline (speedup 1.0000x reference)
import jax
import jax.numpy as jnp
from jax.experimental import pallas as pl


def kernel(x, edge_index, batch, params):
    raise NotImplementedError("write your pallas kernel here")


# speedup vs baseline: 4.5529x; 4.5529x over previous
import functools

import jax
import jax.numpy as jnp
from jax import lax
from jax.experimental import pallas as pl
from jax.experimental.pallas import tpu as pltpu
from jax.experimental.pallas import tpu_sc as plsc

_NC = 2
_NS = 16
_NW = _NC * _NS

_CH = 80
_ZR = 80
_CHK = 512
_EPSBN = 1e-5


def _segment_sum_sc(h, src, dst):
    n, f = h.shape
    e = src.shape[0]
    epw = e // _NW
    assert epw * _NW == e and epw % _CH == 0
    nrc = n // _ZR
    assert nrc * _ZR == n

    mesh = plsc.VectorSubcoreMesh(core_axis_name="c", subcore_axis_name="s")

    @functools.partial(
        pl.kernel,
        out_type=jax.ShapeDtypeStruct((_NC, n, f), jnp.float32),
        mesh=mesh,
        scratch_types=[
            pltpu.VMEM((_CH,), jnp.int32),
            pltpu.VMEM((_CH,), jnp.int32),
            pltpu.VMEM((_CH, f), jnp.float32),
            pltpu.VMEM((_ZR, f), jnp.float32),
            pltpu.VMEM_SHARED((n, f), jnp.float32),
            pltpu.SemaphoreType.DMA,
        ],
    )
    def agg_kernel(h_hbm, src_hbm, dst_hbm, out_hbm,
                   src_v, dst_v, rows_v, zbuf, acc, sem):
        cid = lax.axis_index("c")
        sid = lax.axis_index("s")

        @pl.loop(0, _ZR)
        def _(r):
            @pl.loop(0, f, step=16)
            def _(c):
                zbuf[r, pl.ds(c, 16)] = jnp.zeros((16,), jnp.float32)

        @pl.loop(sid, nrc, step=_NS)
        def _(ci):
            pltpu.sync_copy(zbuf, acc.at[pl.ds(ci * _ZR, _ZR)])

        plsc.subcore_barrier()

        base = (cid * _NS + sid) * epw

        @pl.loop(0, epw, step=_CH)
        def _(i):
            off = base + i
            pltpu.sync_copy(src_hbm.at[pl.ds(off, _CH)], src_v)
            pltpu.sync_copy(dst_hbm.at[pl.ds(off, _CH)], dst_v)
            pltpu.async_copy(h_hbm.at[src_v], rows_v, sem).wait()
            pltpu.sync_copy(rows_v, acc.at[dst_v], add=True)

        plsc.subcore_barrier()

        @pl.loop(sid, nrc, step=_NS)
        def _(ci):
            pltpu.sync_copy(acc.at[pl.ds(ci * _ZR, _ZR)],
                            out_hbm.at[cid, pl.ds(ci * _ZR, _ZR)])

    return agg_kernel(h, src, dst)


def _bn(z, g, b):
    m = jnp.mean(z, axis=0, keepdims=True)
    c = z - m
    v = jnp.mean(c * c, axis=0, keepdims=True)
    return c * lax.rsqrt(v + _EPSBN) * g + b


def _gelu(z):
    return 0.5 * z * (1.0 + lax.erf(z * 0.7071067811865476))


def _bn_starts_body(x_ref, g_ref, b_ref, batch_ref, h_ref, starts_ref):
    x = x_ref[...]
    h_ref[...] = _bn(x, g_ref[...], b_ref[...])
    gids = lax.broadcasted_iota(jnp.int32, (1, 128), 1)
    cmp = (batch_ref[...] < gids).astype(jnp.int32)
    starts_ref[...] = jnp.sum(cmp, axis=0, keepdims=True)


def _mlp_body(h_ref, p0_ref, p1_ref, eps_ref, w1_ref, b1_ref,
              mg_ref, mb_ref, w2_ref, b2_ref, g_ref, bb_ref, o_ref):
    z = (1.0 + eps_ref[...]) * h_ref[...] + (p0_ref[...] + p1_ref[...])
    z = jnp.dot(z, w1_ref[...], preferred_element_type=jnp.float32) + b1_ref[...]
    z = _gelu(_bn(z, mg_ref[...], mb_ref[...]))
    z = jnp.dot(z, w2_ref[...], preferred_element_type=jnp.float32) + b2_ref[...]
    o_ref[...] = _gelu(_bn(z, g_ref[...], bb_ref[...]))


def _final_body(starts_sref, z1_ref, z2_ref, z3_ref, attw_ref, pw_ref,
                w1_ref, b1_ref, lng_ref, lnb_ref, w2_ref, b2_ref,
                out_ref, xj_s, sum_s, max_s, cnt_s):
    n = z1_ref.shape[0]
    f = z1_ref.shape[1]
    z1, z2, z3 = z1_ref[...], z2_ref[...], z3_ref[...]

    inv_f = 1.0 / f
    s1 = jnp.sum(z1 * attw_ref[0:1, :], axis=1, keepdims=True) * inv_f
    s2 = jnp.sum(z2 * attw_ref[1:2, :], axis=1, keepdims=True) * inv_f
    s3 = jnp.sum(z3 * attw_ref[2:3, :], axis=1, keepdims=True) * inv_f
    m = jnp.maximum(jnp.maximum(s1, s2), s3)
    e1 = jnp.exp(s1 - m)
    e2 = jnp.exp(s2 - m)
    e3 = jnp.exp(s3 - m)
    inv = 1.0 / (e1 + e2 + e3)
    xj_s[0:n, :] = (e1 * z1 + e2 * z2 + e3 * z3) * inv
    xj_s[pl.ds(n, _CHK), :] = jnp.zeros((_CHK, f), jnp.float32)

    ng = out_ref.shape[0]

    def pool_g(g, _):
        start = starts_sref[0, g]
        end = starts_sref[0, g + 1]

        def cond(c):
            return c[0] < end

        def body(c):
            pos, s, mx = c
            rows = xj_s[pl.ds(pos, _CHK), :]
            ridx = lax.broadcasted_iota(jnp.int32, (_CHK, 1), 0) + pos
            keep = ridx < end
            s = s + jnp.sum(jnp.where(keep, rows, 0.0), axis=0, keepdims=True)
            mx = jnp.maximum(
                mx, jnp.max(jnp.where(keep, rows, -jnp.inf), axis=0,
                            keepdims=True))
            return pos + _CHK, s, mx

        _, s, mx = lax.while_loop(
            cond, body,
            (start, jnp.zeros((1, f), jnp.float32),
             jnp.full((1, f), -jnp.inf, jnp.float32)))
        sum_s[pl.ds(g, 1), :] = s
        max_s[pl.ds(g, 1), :] = mx
        cnt_s[pl.ds(g, 1), :] = jnp.full((1, f), (end - start).astype(jnp.float32))
        return 0

    lax.fori_loop(0, ng, pool_g, 0)

    addp = sum_s[...]
    cnt = cnt_s[...]
    meanp = addp / jnp.maximum(cnt, 1.0)
    maxp = jnp.where(cnt > 0.0, max_s[...], 0.0)

    p = pw_ref[...]
    pe = jnp.exp(p - jnp.max(p))
    pw = pe / jnp.sum(pe)
    pooled = (addp * pw[:, 0:1] + meanp * pw[:, 1:2] + maxp * pw[:, 2:3])

    o = jnp.dot(pooled, w1_ref[...], preferred_element_type=jnp.float32) + b1_ref[...]
    mu = jnp.mean(o, axis=1, keepdims=True)
    c = o - mu
    v = jnp.mean(c * c, axis=1, keepdims=True)
    o = c * lax.rsqrt(v + _EPSBN) * lng_ref[...] + lnb_ref[...]
    o = _gelu(o) + pooled
    out_ref[...] = jnp.dot(o, w2_ref[...], preferred_element_type=jnp.float32) + b2_ref[...]


def _vmem():
    return pl.BlockSpec(memory_space=pltpu.ANY)


@jax.jit
def kernel(x, edge_index, batch, params):
    n, d = x.shape
    h_dim = params["fc1_w0"].shape[0]
    lat = params["fc2_w"].shape[0]
    num_l = 3
    g_num = 64

    src = edge_index[0]
    dst = edge_index[1]
    batch_col = batch.reshape(n, 1).astype(jnp.int32)

    vspec = pl.BlockSpec(memory_space=pltpu.VMEM)

    h0, starts = pl.pallas_call(
        _bn_starts_body,
        out_shape=(jax.ShapeDtypeStruct((n, d), jnp.float32),
                   jax.ShapeDtypeStruct((1, 128), jnp.int32)),
        in_specs=[vspec] * 4,
        out_specs=(vspec, vspec),
    )(x, params["bn_in_g"].reshape(1, d), params["bn_in_b"].reshape(1, d),
      batch_col)

    mlp = pl.pallas_call(
        _mlp_body,
        out_shape=jax.ShapeDtypeStruct((n, h_dim), jnp.float32),
        in_specs=[vspec] * 12,
        out_specs=vspec,
    )

    h = h0
    hidden = []
    for l in range(num_l):
        parts = _segment_sum_sc(h, src, dst)
        h = mlp(h, parts[0], parts[1],
                params["eps%d" % l].reshape(1, 1),
                params["fc1_w%d" % l].T,
                params["fc1_b%d" % l].reshape(1, h_dim),
                params["mbn_g%d" % l].reshape(1, h_dim),
                params["mbn_b%d" % l].reshape(1, h_dim),
                params["fc2_w%d" % l].T,
                params["fc2_b%d" % l].reshape(1, h_dim),
                params["bn_g%d" % l].reshape(1, h_dim),
                params["bn_b%d" % l].reshape(1, h_dim))
        hidden.append(h)

    smem_spec = pl.BlockSpec(memory_space=pltpu.SMEM)
    out = pl.pallas_call(
        _final_body,
        out_shape=jax.ShapeDtypeStruct((g_num, lat), jnp.float32),
        in_specs=[smem_spec] + [vspec] * 11,
        out_specs=vspec,
        scratch_shapes=[
            pltpu.VMEM((n + _CHK, h_dim), jnp.float32),
            pltpu.VMEM((g_num, h_dim), jnp.float32),
            pltpu.VMEM((g_num, h_dim), jnp.float32),
            pltpu.VMEM((g_num, h_dim), jnp.float32),
        ],
    )(starts, hidden[0], hidden[1], hidden[2],
      params["att_w"], params["pool_w"].reshape(1, 3),
      params["fc1_w"].T, params["fc1_b"].reshape(1, h_dim),
      params["ln_g"].reshape(1, h_dim), params["ln_b"].reshape(1, h_dim),
      params["fc2_w"].T, params["fc2_b"].reshape(1, lat))
    return out


# speedup vs baseline: 7.1118x; 1.5620x over previous
import functools

import jax
import jax.numpy as jnp
from jax import lax
from jax.experimental import pallas as pl
from jax.experimental.pallas import tpu as pltpu
from jax.experimental.pallas import tpu_sc as plsc

_NC = 2
_NS = 16
_NW = _NC * _NS

_CH = 80
_ZR = 80
_CHK = 512
_EPSBN = 1e-5


def _segment_sum_sc(h, src, dst):
    n, f = h.shape
    e = src.shape[0]
    epw = e // _NW
    assert epw * _NW == e and epw % _CH == 0
    nrc = n // _ZR
    assert nrc * _ZR == n

    mesh = plsc.VectorSubcoreMesh(core_axis_name="c", subcore_axis_name="s")

    @functools.partial(
        pl.kernel,
        out_type=jax.ShapeDtypeStruct((_NC, n, f), jnp.float32),
        mesh=mesh,
        scratch_types=[
            pltpu.VMEM((2, _CH), jnp.int32),
            pltpu.VMEM((2, _CH), jnp.int32),
            pltpu.VMEM((2, _CH, f), jnp.float32),
            pltpu.VMEM((_ZR, f), jnp.float32),
            pltpu.VMEM_SHARED((n, f), jnp.float32),
            pltpu.SemaphoreType.DMA,
            pltpu.SemaphoreType.DMA,
        ],
    )
    def agg_kernel(h_hbm, src_hbm, dst_hbm, out_hbm,
                   src_v, dst_v, rows_v, zbuf, acc, sem0, sem1):
        cid = lax.axis_index("c")
        sid = lax.axis_index("s")
        sems = (sem0, sem1)

        @pl.loop(0, _ZR)
        def _(r):
            @pl.loop(0, f, step=16)
            def _(c):
                zbuf[r, pl.ds(c, 16)] = jnp.zeros((16,), jnp.float32)

        @pl.loop(sid, nrc, step=_NS)
        def _(ci):
            pltpu.sync_copy(zbuf, acc.at[pl.ds(ci * _ZR, _ZR)])

        plsc.subcore_barrier()

        base = (cid * _NS + sid) * epw
        nch = epw // _CH

        def load_idx(ci, b):
            off = base + ci * _CH
            pltpu.sync_copy(src_hbm.at[pl.ds(off, _CH)], src_v.at[b])
            pltpu.sync_copy(dst_hbm.at[pl.ds(off, _CH)], dst_v.at[b])

        def start_gather(b):
            pltpu.async_copy(h_hbm.at[src_v.at[b]], rows_v.at[b], sems[b])

        def wait_gather(b):
            pltpu.make_async_copy(h_hbm.at[src_v.at[b]], rows_v.at[b],
                                  sems[b]).wait()

        def scatter(b):
            pltpu.sync_copy(rows_v.at[b], acc.at[dst_v.at[b]], add=True)

        load_idx(0, 0)
        start_gather(0)

        @pl.loop(0, nch - 1, step=2)
        def _(i):
            for j in range(2):
                load_idx(i + j + 1, 1 - j)
                start_gather(1 - j)
                wait_gather(j)
                scatter(j)

        b_last = (nch - 1) % 2
        wait_gather(b_last)
        scatter(b_last)

        plsc.subcore_barrier()

        @pl.loop(sid, nrc, step=_NS)
        def _(ci):
            pltpu.sync_copy(acc.at[pl.ds(ci * _ZR, _ZR)],
                            out_hbm.at[cid, pl.ds(ci * _ZR, _ZR)])

    return agg_kernel(h, src, dst)


def _bn(z, g, b):
    m = jnp.mean(z, axis=0, keepdims=True)
    c = z - m
    v = jnp.mean(c * c, axis=0, keepdims=True)
    return c * lax.rsqrt(v + _EPSBN) * g + b


def _gelu(z):
    return 0.5 * z * (1.0 + lax.erf(z * 0.7071067811865476))


def _bn_starts_body(x_ref, g_ref, b_ref, batch_ref, h_ref, starts_ref):
    x = x_ref[...]
    h_ref[...] = _bn(x, g_ref[...], b_ref[...])
    gids = lax.broadcasted_iota(jnp.int32, (1, 128), 1)
    cmp = (batch_ref[...] < gids).astype(jnp.int32)
    starts_ref[...] = jnp.sum(cmp, axis=0, keepdims=True)


def _mlp_body(h_ref, p0_ref, p1_ref, eps_ref, w1_ref, b1_ref,
              mg_ref, mb_ref, w2_ref, b2_ref, g_ref, bb_ref, o_ref):
    z = (1.0 + eps_ref[...]) * h_ref[...] + (p0_ref[...] + p1_ref[...])
    z = jnp.dot(z, w1_ref[...], preferred_element_type=jnp.float32) + b1_ref[...]
    z = _gelu(_bn(z, mg_ref[...], mb_ref[...]))
    z = jnp.dot(z, w2_ref[...], preferred_element_type=jnp.float32) + b2_ref[...]
    o_ref[...] = _gelu(_bn(z, g_ref[...], bb_ref[...]))


def _final_body(starts_sref, z1_ref, z2_ref, z3_ref, attw_ref, pw_ref,
                w1_ref, b1_ref, lng_ref, lnb_ref, w2_ref, b2_ref,
                out_ref, xj_s, sum_s, max_s, cnt_s):
    n = z1_ref.shape[0]
    f = z1_ref.shape[1]
    z1, z2, z3 = z1_ref[...], z2_ref[...], z3_ref[...]

    inv_f = 1.0 / f
    s1 = jnp.sum(z1 * attw_ref[0:1, :], axis=1, keepdims=True) * inv_f
    s2 = jnp.sum(z2 * attw_ref[1:2, :], axis=1, keepdims=True) * inv_f
    s3 = jnp.sum(z3 * attw_ref[2:3, :], axis=1, keepdims=True) * inv_f
    m = jnp.maximum(jnp.maximum(s1, s2), s3)
    e1 = jnp.exp(s1 - m)
    e2 = jnp.exp(s2 - m)
    e3 = jnp.exp(s3 - m)
    inv = 1.0 / (e1 + e2 + e3)
    xj_s[0:n, :] = (e1 * z1 + e2 * z2 + e3 * z3) * inv
    xj_s[pl.ds(n, _CHK), :] = jnp.zeros((_CHK, f), jnp.float32)

    ng = out_ref.shape[0]

    def pool_g(g, _):
        start = starts_sref[0, g]
        end = starts_sref[0, g + 1]

        def cond(c):
            return c[0] < end

        def body(c):
            pos, s, mx = c
            rows = xj_s[pl.ds(pos, _CHK), :]
            ridx = lax.broadcasted_iota(jnp.int32, (_CHK, 1), 0) + pos
            keep = ridx < end
            s = s + jnp.sum(jnp.where(keep, rows, 0.0), axis=0, keepdims=True)
            mx = jnp.maximum(
                mx, jnp.max(jnp.where(keep, rows, -jnp.inf), axis=0,
                            keepdims=True))
            return pos + _CHK, s, mx

        _, s, mx = lax.while_loop(
            cond, body,
            (start, jnp.zeros((1, f), jnp.float32),
             jnp.full((1, f), -jnp.inf, jnp.float32)))
        sum_s[pl.ds(g, 1), :] = s
        max_s[pl.ds(g, 1), :] = mx
        cnt_s[pl.ds(g, 1), :] = jnp.full((1, f), (end - start).astype(jnp.float32))
        return 0

    lax.fori_loop(0, ng, pool_g, 0)

    addp = sum_s[...]
    cnt = cnt_s[...]
    meanp = addp / jnp.maximum(cnt, 1.0)
    maxp = jnp.where(cnt > 0.0, max_s[...], 0.0)

    p = pw_ref[...]
    pe = jnp.exp(p - jnp.max(p))
    pw = pe / jnp.sum(pe)
    pooled = (addp * pw[:, 0:1] + meanp * pw[:, 1:2] + maxp * pw[:, 2:3])

    o = jnp.dot(pooled, w1_ref[...], preferred_element_type=jnp.float32) + b1_ref[...]
    mu = jnp.mean(o, axis=1, keepdims=True)
    c = o - mu
    v = jnp.mean(c * c, axis=1, keepdims=True)
    o = c * lax.rsqrt(v + _EPSBN) * lng_ref[...] + lnb_ref[...]
    o = _gelu(o) + pooled
    out_ref[...] = jnp.dot(o, w2_ref[...], preferred_element_type=jnp.float32) + b2_ref[...]


def _vmem():
    return pl.BlockSpec(memory_space=pltpu.ANY)


@jax.jit
def kernel(x, edge_index, batch, params):
    n, d = x.shape
    h_dim = params["fc1_w0"].shape[0]
    lat = params["fc2_w"].shape[0]
    num_l = 3
    g_num = 64

    src = edge_index[0]
    dst = edge_index[1]
    batch_col = batch.reshape(n, 1).astype(jnp.int32)

    vspec = pl.BlockSpec(memory_space=pltpu.VMEM)

    h0, starts = pl.pallas_call(
        _bn_starts_body,
        out_shape=(jax.ShapeDtypeStruct((n, d), jnp.float32),
                   jax.ShapeDtypeStruct((1, 128), jnp.int32)),
        in_specs=[vspec] * 4,
        out_specs=(vspec, vspec),
    )(x, params["bn_in_g"].reshape(1, d), params["bn_in_b"].reshape(1, d),
      batch_col)

    mlp = pl.pallas_call(
        _mlp_body,
        out_shape=jax.ShapeDtypeStruct((n, h_dim), jnp.float32),
        in_specs=[vspec] * 12,
        out_specs=vspec,
    )

    h = h0
    hidden = []
    for l in range(num_l):
        parts = _segment_sum_sc(h, src, dst)
        h = mlp(h, parts[0], parts[1],
                params["eps%d" % l].reshape(1, 1),
                params["fc1_w%d" % l].T,
                params["fc1_b%d" % l].reshape(1, h_dim),
                params["mbn_g%d" % l].reshape(1, h_dim),
                params["mbn_b%d" % l].reshape(1, h_dim),
                params["fc2_w%d" % l].T,
                params["fc2_b%d" % l].reshape(1, h_dim),
                params["bn_g%d" % l].reshape(1, h_dim),
                params["bn_b%d" % l].reshape(1, h_dim))
        hidden.append(h)

    smem_spec = pl.BlockSpec(memory_space=pltpu.SMEM)
    out = pl.pallas_call(
        _final_body,
        out_shape=jax.ShapeDtypeStruct((g_num, lat), jnp.float32),
        in_specs=[smem_spec] + [vspec] * 11,
        out_specs=vspec,
        scratch_shapes=[
            pltpu.VMEM((n + _CHK, h_dim), jnp.float32),
            pltpu.VMEM((g_num, h_dim), jnp.float32),
            pltpu.VMEM((g_num, h_dim), jnp.float32),
            pltpu.VMEM((g_num, h_dim), jnp.float32),
        ],
    )(starts, hidden[0], hidden[1], hidden[2],
      params["att_w"], params["pool_w"].reshape(1, 3),
      params["fc1_w"].T, params["fc1_b"].reshape(1, h_dim),
      params["ln_g"].reshape(1, h_dim), params["ln_b"].reshape(1, h_dim),
      params["fc2_w"].T, params["fc2_b"].reshape(1, lat))
    return out
